# Initial kernel scaffold; baseline (speedup 1.0000x reference)
import jax
import jax.numpy as jnp
from jax.experimental import pallas as pl


def kernel(x, batch, size, W):
    raise NotImplementedError("write your pallas kernel here")


# speedup vs baseline: 2.2506x; 2.2506x over previous
import functools

import jax
import jax.numpy as jnp
from jax import lax
from jax.experimental import pallas as pl
from jax.experimental.pallas import tpu as pltpu
from jax.experimental.pallas import tpu_sc as plsc

N = 100000
D = 512
S = 512
NL = 16
NV = D // NL
CH = 80
ROWS_W = 3120
NCHUNK = ROWS_W // CH
NCHUNK_LAST = (N - 31 * ROWS_W) // CH
NW = 32


def _lane_sum(v):
    lanes = lax.iota(jnp.int32, NL)
    for k in (1, 2, 4, 8):
        v = v + v.at[lanes ^ k].get(mode="promise_in_bounds")
    return v


def _mesh():
    return plsc.VectorSubcoreMesh(core_axis_name="c", subcore_axis_name="s")


def _worker():
    cid = lax.axis_index("c")
    sid = lax.axis_index("s")
    wid = cid * 16 + sid
    base = wid * ROWS_W
    nchunks = jnp.where(wid == NW - 1, NCHUNK_LAST, NCHUNK)
    return wid, base, nchunks


def _sc_pass1_call(x, batch, zero16):
    @functools.partial(
        pl.kernel,
        out_type=(
            jax.ShapeDtypeStruct((NW, S, D), jnp.float32),
            jax.ShapeDtypeStruct((NW, S * NL), jnp.float32),
        ),
        mesh=_mesh(),
        scratch_types=[
            pltpu.VMEM((CH, D), jnp.float32),
            pltpu.VMEM((CH + NL,), jnp.int32),
            pltpu.VMEM((D,), jnp.float32),
            pltpu.VMEM((S * NL,), jnp.float32),
        ],
    )
    def k(x_hbm, b_hbm, z16_hbm, psum_hbm, pcnt_hbm, xbuf, idxbuf, stg, cntv):
        wid, base, nchunks = _worker()
        pltpu.sync_copy(z16_hbm, cntv)

        zero = jnp.zeros((NL,), jnp.float32)

        def flush(prev_id, accs, cnt):
            for c in range(NV):
                stg[pl.ds(c * NL, NL)] = accs[c]
            cntv[pl.ds(prev_id * NL, NL)] = cnt
            pltpu.sync_copy(stg, psum_hbm.at[wid, prev_id])

        def chunk(kk, carry):
            rowbase = base + kk * CH
            pltpu.sync_copy(x_hbm.at[pl.ds(rowbase, CH)], xbuf)
            pltpu.sync_copy(b_hbm.at[pl.ds(rowbase, CH)],
                            idxbuf.at[pl.ds(0, CH)])

            def row(r, rcarry):
                prev_id, cnt, *accs = rcarry
                rid = idxbuf[pl.ds(r, NL)][0]
                change = rid != prev_id

                @pl.when(jnp.logical_and(change, prev_id >= 0))
                def _():
                    flush(prev_id, accs, cnt)

                xs = [xbuf[r, pl.ds(c * NL, NL)] for c in range(NV)]
                new_accs = [
                    jnp.where(change, xs[c], accs[c] + xs[c])
                    for c in range(NV)
                ]
                new_cnt = jnp.where(change, 1.0, cnt + 1.0)
                return (rid, new_cnt, *new_accs)

            return lax.fori_loop(0, CH, row, carry)

        init = (jnp.int32(-1), jnp.zeros((NL,), jnp.float32)) + tuple(
            zero for _ in range(NV))
        prev_id, cnt, *accs = lax.fori_loop(0, nchunks, chunk, init)

        @pl.when(prev_id >= 0)
        def _():
            flush(prev_id, accs, cnt)

        pltpu.sync_copy(cntv, pcnt_hbm.at[wid])

    return k(x, batch, zero16)


def _tc_dense_call(psum, pcnt, W):
    def body(psum_ref, pcnt_ref, w_ref, t_ref, acc, cntacc):
        t = pl.program_id(0)

        @pl.when(t == 0)
        def _():
            acc[...] = jnp.zeros_like(acc)
            cntacc[...] = jnp.zeros_like(cntacc)

        c = pcnt_ref[0, :, 0]
        valid = c > 0.0
        acc[...] += jnp.where(valid[:, None], psum_ref[0], 0.0)
        cntacc[...] += jnp.where(valid, c, 0.0)

        @pl.when(t == NW - 1)
        def _():
            cnt = jnp.maximum(cntacc[...], 1.0)
            mean = acc[...] / cnt[:, None]
            t_ref[...] = jnp.tanh(
                jnp.dot(mean, w_ref[...], preferred_element_type=jnp.float32))

    return pl.pallas_call(
        body,
        grid=(NW,),
        in_specs=[
            pl.BlockSpec((1, S, D), lambda t: (t, 0, 0)),
            pl.BlockSpec((1, S, NL), lambda t: (t, 0, 0)),
            pl.BlockSpec((S, D), lambda t: (0, 0)),
        ],
        out_specs=pl.BlockSpec((S, D), lambda t: (0, 0)),
        out_shape=jax.ShapeDtypeStruct((S, D), jnp.float32),
        scratch_shapes=[
            pltpu.VMEM((S, D), jnp.float32),
            pltpu.VMEM((S,), jnp.float32),
        ],
    )(psum, pcnt, W)


def _sc_pass2_call(x, batch, t):
    @functools.partial(
        pl.kernel,
        out_type=jax.ShapeDtypeStruct((NW, S, D), jnp.float32),
        mesh=_mesh(),
        scratch_types=[
            pltpu.VMEM((CH, D), jnp.float32),
            pltpu.VMEM((CH + NL,), jnp.int32),
            pltpu.VMEM((D,), jnp.float32),
            pltpu.VMEM((D,), jnp.float32),
        ],
    )
    def k(x_hbm, b_hbm, t_hbm, out_hbm, xbuf, idxbuf, stg, trow):
        wid, base, nchunks = _worker()

        def flush(prev_id, accs):
            for c in range(NV):
                stg[pl.ds(c * NL, NL)] = accs[c]
            pltpu.sync_copy(stg, out_hbm.at[wid, prev_id])

        def chunk(kk, carry):
            rowbase = base + kk * CH
            pltpu.sync_copy(x_hbm.at[pl.ds(rowbase, CH)], xbuf)
            pltpu.sync_copy(b_hbm.at[pl.ds(rowbase, CH)],
                            idxbuf.at[pl.ds(0, CH)])

            def row(r, rcarry):
                prev_id, *accs = rcarry
                rid = idxbuf[pl.ds(r, NL)][0]
                change = rid != prev_id

                @pl.when(jnp.logical_and(change, prev_id >= 0))
                def _():
                    flush(prev_id, accs)

                @pl.when(change)
                def _():
                    pltpu.sync_copy(t_hbm.at[rid], trow)

                dacc = (xbuf[r, pl.ds(0, NL)] * trow[pl.ds(0, NL)])
                for c in range(1, NV):
                    dacc = dacc + (xbuf[r, pl.ds(c * NL, NL)]
                                   * trow[pl.ds(c * NL, NL)])
                zv = _lane_sum(dacc * (-10.0))
                cf = 1.0 / (1.0 + jnp.exp(zv))
                new_accs = []
                for c in range(NV):
                    wx = xbuf[r, pl.ds(c * NL, NL)] * cf
                    new_accs.append(jnp.where(change, wx, accs[c] + wx))
                return (rid, *new_accs)

            return lax.fori_loop(0, CH, row, carry)

        init = (jnp.int32(-1),) + tuple(
            jnp.zeros((NL,), jnp.float32) for _ in range(NV))
        prev_id, *accs = lax.fori_loop(0, nchunks, chunk, init)

        @pl.when(prev_id >= 0)
        def _():
            flush(prev_id, accs)

    return k(x, batch, t)


def _tc_combine_call(parts, pcnt):
    def body(parts_ref, pcnt_ref, out_ref, acc):
        t = pl.program_id(0)

        @pl.when(t == 0)
        def _():
            acc[...] = jnp.zeros_like(acc)

        valid = pcnt_ref[0, :, 0] > 0.0
        acc[...] += jnp.where(valid[:, None], parts_ref[0], 0.0)

        @pl.when(t == NW - 1)
        def _():
            out_ref[...] = acc[...]

    return pl.pallas_call(
        body,
        grid=(NW,),
        in_specs=[
            pl.BlockSpec((1, S, D), lambda t: (t, 0, 0)),
            pl.BlockSpec((1, S, NL), lambda t: (t, 0, 0)),
        ],
        out_specs=pl.BlockSpec((S, D), lambda t: (0, 0)),
        out_shape=jax.ShapeDtypeStruct((S, D), jnp.float32),
        scratch_shapes=[pltpu.VMEM((S, D), jnp.float32)],
    )(parts, pcnt)


def kernel(x, batch, size, W):
    del size
    batch = batch.astype(jnp.int32)
    zero16 = jnp.zeros((S * NL,), jnp.float32)
    psum, pcnt = _sc_pass1_call(x, batch, zero16)
    pcnt = pcnt.reshape(NW, S, NL)
    t = _tc_dense_call(psum, pcnt, W)
    parts = _sc_pass2_call(x, batch, t)
    return _tc_combine_call(parts, pcnt)
